# baseline (speedup 1.0000x reference)
import functools

import jax
import jax.numpy as jnp
from jax import lax
from jax.experimental import pallas as pl
from jax.experimental.pallas import tpu as pltpu
from jax.experimental.pallas import tpu_sc as plsc

_B, _L, _D = 512, 512, 256
_G = 16
_NC, _NS = 2, 16
_VPW = _B // (_NC * _NS)


def _tc_body(mask_sref, y_hbm, z_hbm, yi_ref, zi_ref, yo_ref, zo_ref, sems):
    b = pl.program_id(0)
    base = b * _G

    for j in range(_G):
        @pl.when(mask_sref[base + j] == 0)
        def _(j=j):
            pltpu.make_async_copy(y_hbm.at[pl.ds(base + j, 1)],
                                  yo_ref.at[pl.ds(j, 1)], sems.at[0, j]).start()
            pltpu.make_async_copy(z_hbm.at[pl.ds(base + j, 1)],
                                  zo_ref.at[pl.ds(j, 1)], sems.at[1, j]).start()

    yi_row = jnp.broadcast_to(yi_ref[...].reshape(1, 1, _D), (1, _L, _D))
    zi_row = jnp.broadcast_to(zi_ref[...].reshape(1, 1, _D), (1, _L, _D))
    for j in range(_G):
        @pl.when(mask_sref[base + j] != 0)
        def _(j=j):
            yo_ref[pl.ds(j, 1)] = yi_row
            zo_ref[pl.ds(j, 1)] = zi_row

    for j in range(_G):
        @pl.when(mask_sref[base + j] == 0)
        def _(j=j):
            pltpu.make_async_copy(y_hbm.at[pl.ds(base + j, 1)],
                                  yo_ref.at[pl.ds(j, 1)], sems.at[0, j]).wait()
            pltpu.make_async_copy(z_hbm.at[pl.ds(base + j, 1)],
                                  zo_ref.at[pl.ds(j, 1)], sems.at[1, j]).wait()


def _tc_call(y, z, mask_i32, y_init, z_init):
    B, L, D = y.shape
    yi2d = y_init.reshape(1, D)
    zi2d = z_init.reshape(1, D)

    grid_spec = pltpu.PrefetchScalarGridSpec(
        num_scalar_prefetch=1,
        grid=(B // _G,),
        in_specs=[
            pl.BlockSpec(memory_space=pltpu.MemorySpace.HBM),
            pl.BlockSpec(memory_space=pltpu.MemorySpace.HBM),
            pl.BlockSpec((1, D), lambda i, mref: (0, 0)),
            pl.BlockSpec((1, D), lambda i, mref: (0, 0)),
        ],
        out_specs=[
            pl.BlockSpec((_G, L, D), lambda i, mref: (i, 0, 0)),
            pl.BlockSpec((_G, L, D), lambda i, mref: (i, 0, 0)),
        ],
        scratch_shapes=[
            pltpu.SemaphoreType.DMA((2, _G)),
        ],
    )

    y_new, z_new = pl.pallas_call(
        _tc_body,
        grid_spec=grid_spec,
        out_shape=[
            jax.ShapeDtypeStruct((B, L, D), y.dtype),
            jax.ShapeDtypeStruct((B, L, D), z.dtype),
        ],
        compiler_params=pltpu.CompilerParams(
            dimension_semantics=("arbitrary",),
        ),
    )(mask_i32, y, z, yi2d, zi2d)
    return y_new, z_new


def _sc_steps_body(st_hbm, mask_hbm, so_hbm, st_v, m_v, o_v, sem):
    cid = lax.axis_index("c")
    sid = lax.axis_index("s")
    wid = sid * _NC + cid
    base = wid * _VPW

    pltpu.sync_copy(st_hbm.at[pl.ds(base, _VPW)], st_v)
    pltpu.sync_copy(mask_hbm.at[pl.ds(base, _VPW)], m_v)
    o_v[...] = jnp.where(m_v[...] != 0, jnp.zeros_like(st_v[...]), st_v[...])
    pltpu.sync_copy(o_v, so_hbm.at[pl.ds(base, _VPW)])


def _sc_steps_call(steps, mask_i32):
    kfn = functools.partial(
        pl.kernel,
        mesh=plsc.VectorSubcoreMesh(core_axis_name="c", subcore_axis_name="s"),
        out_type=jax.ShapeDtypeStruct((_B,), steps.dtype),
        scratch_types=[
            pltpu.VMEM((_VPW,), jnp.int32),
            pltpu.VMEM((_VPW,), jnp.int32),
            pltpu.VMEM((_VPW,), jnp.int32),
            pltpu.SemaphoreType.DMA,
        ],
    )(_sc_steps_body)
    return kfn(steps, mask_i32)


def kernel(y, z, steps, mask, y_init, z_init):
    mask_i32 = mask.astype(jnp.int32)
    steps_new = _sc_steps_call(steps, mask_i32)
    y_new, z_new = _tc_call(y, z, mask_i32, y_init, z_init)
    return (y_new, z_new, steps_new)


# speedup vs baseline: 1.0009x; 1.0009x over previous
import functools

import jax
import jax.numpy as jnp
from jax import lax
from jax.experimental import pallas as pl
from jax.experimental.pallas import tpu as pltpu
from jax.experimental.pallas import tpu_sc as plsc

_B, _L, _D = 512, 512, 256
_G = 16
_NC, _NS = 2, 16
_VPW = _B // (_NC * _NS)


def _tc_body(mask_sref, y_hbm, z_hbm, yi_ref, zi_ref, yo_ref, zo_ref, sems):
    b = pl.program_id(0)
    base = b * _G

    def xfer(j, w, method):
        getattr(pltpu.make_async_copy(y_hbm.at[pl.ds(base + j, w)],
                                      yo_ref.at[pl.ds(j, w)],
                                      sems.at[0, j]), method)()
        getattr(pltpu.make_async_copy(z_hbm.at[pl.ds(base + j, w)],
                                      zo_ref.at[pl.ds(j, w)],
                                      sems.at[1, j]), method)()

    def sweep(method):
        for j in range(0, _G, 2):
            c0 = mask_sref[base + j] == 0
            c1 = mask_sref[base + j + 1] == 0

            @pl.when(jnp.logical_and(c0, c1))
            def _(j=j):
                xfer(j, 2, method)

            @pl.when(jnp.logical_and(c0, jnp.logical_not(c1)))
            def _(j=j):
                xfer(j, 1, method)

            @pl.when(jnp.logical_and(jnp.logical_not(c0), c1))
            def _(j=j):
                xfer(j + 1, 1, method)

    sweep("start")

    yi_row = jnp.broadcast_to(yi_ref[...].reshape(1, 1, _D), (1, _L, _D))
    zi_row = jnp.broadcast_to(zi_ref[...].reshape(1, 1, _D), (1, _L, _D))
    for j in range(_G):
        @pl.when(mask_sref[base + j] != 0)
        def _(j=j):
            yo_ref[pl.ds(j, 1)] = yi_row
            zo_ref[pl.ds(j, 1)] = zi_row

    sweep("wait")


def _tc_call(y, z, mask_i32, y_init, z_init):
    B, L, D = y.shape
    yi2d = y_init.reshape(1, D)
    zi2d = z_init.reshape(1, D)

    grid_spec = pltpu.PrefetchScalarGridSpec(
        num_scalar_prefetch=1,
        grid=(B // _G,),
        in_specs=[
            pl.BlockSpec(memory_space=pltpu.MemorySpace.HBM),
            pl.BlockSpec(memory_space=pltpu.MemorySpace.HBM),
            pl.BlockSpec((1, D), lambda i, mref: (0, 0)),
            pl.BlockSpec((1, D), lambda i, mref: (0, 0)),
        ],
        out_specs=[
            pl.BlockSpec((_G, L, D), lambda i, mref: (i, 0, 0)),
            pl.BlockSpec((_G, L, D), lambda i, mref: (i, 0, 0)),
        ],
        scratch_shapes=[
            pltpu.SemaphoreType.DMA((2, _G)),
        ],
    )

    y_new, z_new = pl.pallas_call(
        _tc_body,
        grid_spec=grid_spec,
        out_shape=[
            jax.ShapeDtypeStruct((B, L, D), y.dtype),
            jax.ShapeDtypeStruct((B, L, D), z.dtype),
        ],
        compiler_params=pltpu.CompilerParams(
            dimension_semantics=("arbitrary",),
        ),
    )(mask_i32, y, z, yi2d, zi2d)
    return y_new, z_new


def _sc_steps_body(st_hbm, mask_hbm, so_hbm, st_v, m_v, o_v, sem):
    cid = lax.axis_index("c")
    sid = lax.axis_index("s")
    wid = sid * _NC + cid
    base = wid * _VPW

    pltpu.sync_copy(st_hbm.at[pl.ds(base, _VPW)], st_v)
    pltpu.sync_copy(mask_hbm.at[pl.ds(base, _VPW)], m_v)
    o_v[...] = jnp.where(m_v[...] != 0, jnp.zeros_like(st_v[...]), st_v[...])
    pltpu.sync_copy(o_v, so_hbm.at[pl.ds(base, _VPW)])


def _sc_steps_call(steps, mask_i32):
    kfn = functools.partial(
        pl.kernel,
        mesh=plsc.VectorSubcoreMesh(core_axis_name="c", subcore_axis_name="s"),
        out_type=jax.ShapeDtypeStruct((_B,), steps.dtype),
        scratch_types=[
            pltpu.VMEM((_VPW,), jnp.int32),
            pltpu.VMEM((_VPW,), jnp.int32),
            pltpu.VMEM((_VPW,), jnp.int32),
            pltpu.SemaphoreType.DMA,
        ],
    )(_sc_steps_body)
    return kfn(steps, mask_i32)


def kernel(y, z, steps, mask, y_init, z_init):
    mask_i32 = mask.astype(jnp.int32)
    steps_new = _sc_steps_call(steps, mask_i32)
    y_new, z_new = _tc_call(y, z, mask_i32, y_init, z_init)
    return (y_new, z_new, steps_new)
